# baseline (speedup 1.0000x reference)
import functools

import jax
import jax.numpy as jnp
from jax import lax
from jax.experimental import pallas as pl
from jax.experimental.pallas import tpu as pltpu
from jax.experimental.pallas import tpu_sc as plsc

_LANES = 16
_NC = 2
_NS = 16
_NW = _NC * _NS
_CHUNK = 128
_TBLK = 4096


@functools.lru_cache(maxsize=None)
def _build_row_major(V, D):
    grid = (V + _TBLK - 1) // _TBLK

    def body(tt_ref, out_ref):
        eye = jnp.eye(D, D, dtype=jnp.float32)
        xt = lax.dot_general(tt_ref[...], eye, (((0,), (0,)), ((), ())),
                             precision=lax.Precision.HIGHEST,
                             preferred_element_type=jnp.float32)
        out_ref[...] = xt

    return pl.pallas_call(
        body,
        grid=(grid,),
        in_specs=[pl.BlockSpec((D, _TBLK), lambda i: (0, i))],
        out_specs=pl.BlockSpec((_TBLK, D), lambda i: (i, 0)),
        out_shape=jax.ShapeDtypeStruct((V, D), jnp.float32),
    )


@functools.lru_cache(maxsize=None)
def _build_sc_kernel(B, D):
    b_per_w = B // _NW
    n_chunks = b_per_w // _CHUNK
    groups_per_chunk = _CHUNK // _LANES
    n_sub = D // _LANES

    mesh = plsc.VectorSubcoreMesh(core_axis_name="c", subcore_axis_name="s")

    @functools.partial(
        pl.kernel,
        out_type=jax.ShapeDtypeStruct((B,), jnp.float32),
        mesh=mesh,
        compiler_params=pltpu.CompilerParams(needs_layout_passes=False,
                                             use_tc_tiling_on_sc=False),
        scratch_types=[
            pltpu.VMEM((n_chunks, _CHUNK), jnp.int32),
            pltpu.VMEM((n_chunks, _CHUNK), jnp.int32),
            pltpu.VMEM((2, _CHUNK, D), jnp.float32),
            pltpu.VMEM((2, _CHUNK, D), jnp.float32),
            pltpu.VMEM((n_chunks, _CHUNK), jnp.float32),
            pltpu.VMEM((n_chunks, _CHUNK), jnp.float32),
            pltpu.VMEM((b_per_w,), jnp.float32),
            pltpu.VMEM((_LANES,), jnp.float32),
            pltpu.SemaphoreType.DMA,
            pltpu.SemaphoreType.DMA,
            pltpu.SemaphoreType.DMA,
        ],
    )
    def sc_kernel(p1_hbm, p2_hbm, table_hbm, pbias_hbm, bias_hbm, out_hbm,
                  idx1_v, idx2_v, rows1_v, rows2_v, b1_v, b2_v, out_v,
                  bias_v, sem0, sem1, semb):
        wid = lax.axis_index("s") * _NC + lax.axis_index("c")
        base = wid * b_per_w

        pltpu.sync_copy(bias_hbm, bias_v)
        for c in range(n_chunks):
            pltpu.sync_copy(p1_hbm.at[pl.ds(base + c * _CHUNK, _CHUNK)],
                            idx1_v.at[c])
            pltpu.sync_copy(p2_hbm.at[pl.ds(base + c * _CHUNK, _CHUNK)],
                            idx2_v.at[c])

        sems = [sem0, sem1]

        def fire(c):
            s = sems[c % 2]
            return (pltpu.async_copy(table_hbm.at[idx1_v.at[c]],
                                     rows1_v.at[c % 2], s),
                    pltpu.async_copy(table_hbm.at[idx2_v.at[c]],
                                     rows2_v.at[c % 2], s))

        bias_copies = []
        for c in range(n_chunks):
            bias_copies.append(pltpu.async_copy(pbias_hbm.at[idx1_v.at[c]],
                                                b1_v.at[c], semb))
            bias_copies.append(pltpu.async_copy(pbias_hbm.at[idx2_v.at[c]],
                                                b2_v.at[c], semb))

        in_flight = [fire(0), fire(1)]
        for cp in bias_copies:
            cp.wait()

        for c in range(n_chunks):
            for cp in in_flight[0]:
                cp.wait()
            in_flight = in_flight[1:]
            slot = c % 2

            def group_body(g, carry, c=c, slot=slot):
                iota = lax.iota(jnp.int32, _LANES)
                bias_splat = bias_v[...]
                r0 = g * _LANES
                acc = jnp.zeros((_LANES,), jnp.float32)
                for j in range(_LANES):
                    r = r0 + j
                    t = None
                    for q in range(n_sub):
                        a = rows1_v[slot, r, pl.ds(q * _LANES, _LANES)]
                        b = rows2_v[slot, r, pl.ds(q * _LANES, _LANES)]
                        t = a * b if t is None else t + a * b
                    acc = jnp.where(iota == j, jnp.sum(t), acc)
                vb1 = b1_v[c, pl.ds(r0, _LANES)]
                vb2 = b2_v[c, pl.ds(r0, _LANES)]
                x = acc + vb1 + vb2 + bias_splat
                out_v[pl.ds(c * _CHUNK + r0, _LANES)] = (
                    1.0 / (1.0 + jnp.exp(-x)))
                return carry
            lax.fori_loop(0, groups_per_chunk, group_body, 0)

            if c + 2 < n_chunks:
                in_flight.append(fire(c + 2))

        pltpu.sync_copy(out_v, out_hbm.at[pl.ds(base, b_per_w)])

    return sc_kernel


def kernel(product1, product2, product_embedding, product_bias, bias):
    V, D = product_embedding.shape
    table_rm = _build_row_major(V, D)(product_embedding.T)
    sc_kernel = _build_sc_kernel(product1.shape[0], D)
    pbias_flat = jnp.reshape(product_bias, (-1,))
    bias_vec = jnp.broadcast_to(bias, (_LANES,)).astype(jnp.float32)
    return sc_kernel(product1.astype(jnp.int32), product2.astype(jnp.int32),
                     table_rm, pbias_flat, bias_vec)


# speedup vs baseline: 1.3034x; 1.3034x over previous
import functools

import jax
import jax.numpy as jnp
from jax import lax
from jax.experimental import pallas as pl
from jax.experimental.pallas import tpu as pltpu
from jax.experimental.pallas import tpu_sc as plsc

_LANES = 16
_NC = 2
_NS = 16
_NW = _NC * _NS
_CHUNK = 128
_TBLK = 4096


@functools.lru_cache(maxsize=None)
def _build_row_major(V, D):
    grid = (V + _TBLK - 1) // _TBLK

    def body(tt_ref, out_ref):
        eye = jnp.eye(D, D, dtype=jnp.float32)
        xt = lax.dot_general(tt_ref[...], eye, (((0,), (0,)), ((), ())),
                             precision=lax.Precision.HIGHEST,
                             preferred_element_type=jnp.float32)
        out_ref[...] = xt

    return pl.pallas_call(
        body,
        grid=(grid,),
        in_specs=[pl.BlockSpec((D, _TBLK), lambda i: (0, i))],
        out_specs=pl.BlockSpec((_TBLK, D), lambda i: (i, 0)),
        out_shape=jax.ShapeDtypeStruct((V, D), jnp.float32),
    )


@functools.lru_cache(maxsize=None)
def _build_sc_kernel(B, D):
    b_per_w = B // _NW
    n_chunks = b_per_w // _CHUNK
    groups_per_chunk = _CHUNK // _LANES
    n_sub = D // _LANES

    mesh = plsc.VectorSubcoreMesh(core_axis_name="c", subcore_axis_name="s")

    @functools.partial(
        pl.kernel,
        out_type=jax.ShapeDtypeStruct((B,), jnp.float32),
        mesh=mesh,
        compiler_params=pltpu.CompilerParams(needs_layout_passes=False,
                                             use_tc_tiling_on_sc=False),
        scratch_types=[
            pltpu.VMEM((n_chunks, _CHUNK), jnp.int32),
            pltpu.VMEM((n_chunks, _CHUNK), jnp.int32),
            pltpu.VMEM((2, _CHUNK, D), jnp.float32),
            pltpu.VMEM((2, _CHUNK, D), jnp.float32),
            pltpu.VMEM((n_chunks, _CHUNK), jnp.float32),
            pltpu.VMEM((n_chunks, _CHUNK), jnp.float32),
            pltpu.VMEM((b_per_w,), jnp.float32),
            pltpu.VMEM((_LANES,), jnp.float32),
            pltpu.SemaphoreType.DMA,
            pltpu.SemaphoreType.DMA,
            pltpu.SemaphoreType.DMA,
        ],
    )
    def sc_kernel(p1_hbm, p2_hbm, table_hbm, pbias_hbm, bias_hbm, out_hbm,
                  idx1_v, idx2_v, rows1_v, rows2_v, b1_v, b2_v, out_v,
                  bias_v, sem0, sem1, semb):
        wid = lax.axis_index("s") * _NC + lax.axis_index("c")
        base = wid * b_per_w

        pltpu.sync_copy(bias_hbm, bias_v)
        for c in range(n_chunks):
            pltpu.sync_copy(p1_hbm.at[pl.ds(base + c * _CHUNK, _CHUNK)],
                            idx1_v.at[c])
            pltpu.sync_copy(p2_hbm.at[pl.ds(base + c * _CHUNK, _CHUNK)],
                            idx2_v.at[c])

        sems = [sem0, sem1]

        def fire(c):
            s = sems[c % 2]
            return (pltpu.async_copy(table_hbm.at[idx1_v.at[c]],
                                     rows1_v.at[c % 2], s),
                    pltpu.async_copy(table_hbm.at[idx2_v.at[c]],
                                     rows2_v.at[c % 2], s))

        bias_copies = []
        for c in range(n_chunks):
            bias_copies.append(pltpu.async_copy(pbias_hbm.at[idx1_v.at[c]],
                                                b1_v.at[c], semb))
            bias_copies.append(pltpu.async_copy(pbias_hbm.at[idx2_v.at[c]],
                                                b2_v.at[c], semb))

        in_flight = [fire(0), fire(1)]
        for cp in bias_copies:
            cp.wait()

        for c in range(n_chunks):
            for cp in in_flight[0]:
                cp.wait()
            in_flight = in_flight[1:]
            slot = c % 2

            def group_body(g, carry, c=c, slot=slot):
                iota = lax.iota(jnp.int32, _LANES)
                bias_splat = bias_v[...]
                r0 = g * _LANES
                acc = jnp.zeros((_LANES,), jnp.float32)
                for j in range(_LANES):
                    r = r0 + j
                    t = None
                    for q in range(n_sub):
                        a = rows1_v[slot, r, pl.ds(q * _LANES, _LANES)]
                        b = rows2_v[slot, r, pl.ds(q * _LANES, _LANES)]
                        t = a * b if t is None else t + a * b
                    acc = jnp.where(iota == j, jnp.sum(t), acc)
                vb1 = b1_v[c, pl.ds(r0, _LANES)]
                vb2 = b2_v[c, pl.ds(r0, _LANES)]
                x = acc + vb1 + vb2 + bias_splat
                out_v[pl.ds(c * _CHUNK + r0, _LANES)] = (
                    1.0 / (1.0 + jnp.exp(-x)))
                return carry
            lax.fori_loop(0, groups_per_chunk, group_body, 0)

            if c + 2 < n_chunks:
                in_flight.append(fire(c + 2))

        pltpu.sync_copy(out_v, out_hbm.at[pl.ds(base, b_per_w)])

    return sc_kernel


def kernel(product1, product2, product_embedding, product_bias, bias):
    V, D = product_embedding.shape
    table_rm = product_embedding
    sc_kernel = _build_sc_kernel(product1.shape[0], D)
    pbias_flat = jnp.reshape(product_bias, (-1,))
    bias_vec = jnp.broadcast_to(bias, (_LANES,)).astype(jnp.float32)
    return sc_kernel(product1.astype(jnp.int32), product2.astype(jnp.int32),
                     table_rm, pbias_flat, bias_vec)


# speedup vs baseline: 1.3597x; 1.0432x over previous
import functools

import jax
import jax.numpy as jnp
from jax import lax
from jax.experimental import pallas as pl
from jax.experimental.pallas import tpu as pltpu
from jax.experimental.pallas import tpu_sc as plsc

_LANES = 16
_NC = 2
_NS = 16
_NW = _NC * _NS
_CHUNK = 128
_TBLK = 4096


@functools.lru_cache(maxsize=None)
def _build_row_major(V, D):
    grid = (V + _TBLK - 1) // _TBLK

    def body(tt_ref, out_ref):
        eye = jnp.eye(D, D, dtype=jnp.float32)
        xt = lax.dot_general(tt_ref[...], eye, (((0,), (0,)), ((), ())),
                             precision=lax.Precision.HIGHEST,
                             preferred_element_type=jnp.float32)
        out_ref[...] = xt

    return pl.pallas_call(
        body,
        grid=(grid,),
        in_specs=[pl.BlockSpec((D, _TBLK), lambda i: (0, i))],
        out_specs=pl.BlockSpec((_TBLK, D), lambda i: (i, 0)),
        out_shape=jax.ShapeDtypeStruct((V, D), jnp.float32),
    )


@functools.lru_cache(maxsize=None)
def _build_sc_kernel(B, D):
    b_per_w = B // _NW
    n_chunks = b_per_w // _CHUNK
    groups_per_chunk = _CHUNK // _LANES
    n_sub = D // _LANES

    mesh = plsc.VectorSubcoreMesh(core_axis_name="c", subcore_axis_name="s")

    @functools.partial(
        pl.kernel,
        out_type=jax.ShapeDtypeStruct((B,), jnp.float32),
        mesh=mesh,
        compiler_params=pltpu.CompilerParams(needs_layout_passes=False,
                                             use_tc_tiling_on_sc=False),
        scratch_types=[
            pltpu.VMEM((n_chunks, _CHUNK), jnp.int32),
            pltpu.VMEM((n_chunks, _CHUNK), jnp.int32),
            pltpu.VMEM((2, _CHUNK, D), jnp.float32),
            pltpu.VMEM((2, _CHUNK, D), jnp.float32),
            pltpu.VMEM((n_chunks, _CHUNK), jnp.float32),
            pltpu.VMEM((n_chunks, _CHUNK), jnp.float32),
            pltpu.VMEM((b_per_w,), jnp.float32),
            pltpu.VMEM((_LANES,), jnp.float32),
            pltpu.SemaphoreType.DMA,
            pltpu.SemaphoreType.DMA,
            pltpu.SemaphoreType.DMA,
            pltpu.SemaphoreType.DMA,
        ],
    )
    def sc_kernel(p1_hbm, p2_hbm, table_hbm, pbias_hbm, bias_hbm, out_hbm,
                  idx1_v, idx2_v, rows1_v, rows2_v, b1_v, b2_v, out_v,
                  bias_v, sem0, sem1, semb, semi):
        wid = lax.axis_index("s") * _NC + lax.axis_index("c")
        base = wid * b_per_w

        idx_copies = []
        for c in range(n_chunks):
            idx_copies.append(
                pltpu.async_copy(p1_hbm.at[pl.ds(base + c * _CHUNK, _CHUNK)],
                                 idx1_v.at[c], semi))
            idx_copies.append(
                pltpu.async_copy(p2_hbm.at[pl.ds(base + c * _CHUNK, _CHUNK)],
                                 idx2_v.at[c], semi))
        pltpu.sync_copy(bias_hbm, bias_v)
        for cp in idx_copies:
            cp.wait()

        sems = [sem0, sem1]

        def fire(c):
            s = sems[c % 2]
            return (pltpu.async_copy(table_hbm.at[idx1_v.at[c]],
                                     rows1_v.at[c % 2], s),
                    pltpu.async_copy(table_hbm.at[idx2_v.at[c]],
                                     rows2_v.at[c % 2], s))

        in_flight = [fire(0), fire(1)]

        bias_copies = []
        for c in range(n_chunks):
            bias_copies.append((pltpu.async_copy(pbias_hbm.at[idx1_v.at[c]],
                                                 b1_v.at[c], semb),
                                pltpu.async_copy(pbias_hbm.at[idx2_v.at[c]],
                                                 b2_v.at[c], semb)))

        for c in range(n_chunks):
            for cp in bias_copies[c]:
                cp.wait()
            for cp in in_flight[0]:
                cp.wait()
            in_flight = in_flight[1:]
            slot = c % 2

            def group_body(g, carry, c=c, slot=slot):
                iota = lax.iota(jnp.int32, _LANES)
                bias_splat = bias_v[...]
                r0 = g * _LANES
                acc = jnp.zeros((_LANES,), jnp.float32)
                for j in range(_LANES):
                    r = r0 + j
                    t = None
                    for q in range(n_sub):
                        a = rows1_v[slot, r, pl.ds(q * _LANES, _LANES)]
                        b = rows2_v[slot, r, pl.ds(q * _LANES, _LANES)]
                        t = a * b if t is None else t + a * b
                    acc = jnp.where(iota == j, jnp.sum(t), acc)
                vb1 = b1_v[c, pl.ds(r0, _LANES)]
                vb2 = b2_v[c, pl.ds(r0, _LANES)]
                x = acc + vb1 + vb2 + bias_splat
                out_v[pl.ds(c * _CHUNK + r0, _LANES)] = (
                    1.0 / (1.0 + jnp.exp(-x)))
                return carry
            lax.fori_loop(0, groups_per_chunk, group_body, 0)

            if c + 2 < n_chunks:
                in_flight.append(fire(c + 2))

        pltpu.sync_copy(out_v, out_hbm.at[pl.ds(base, b_per_w)])

    return sc_kernel


def kernel(product1, product2, product_embedding, product_bias, bias):
    V, D = product_embedding.shape
    table_rm = product_embedding
    sc_kernel = _build_sc_kernel(product1.shape[0], D)
    pbias_flat = jnp.reshape(product_bias, (-1,))
    bias_vec = jnp.broadcast_to(bias, (_LANES,)).astype(jnp.float32)
    return sc_kernel(product1.astype(jnp.int32), product2.astype(jnp.int32),
                     table_rm, pbias_flat, bias_vec)
